# Initial kernel scaffold; baseline (speedup 1.0000x reference)
import jax
import jax.numpy as jnp
from jax.experimental import pallas as pl


def kernel(x, conv0_w, conv0_b, conv1_w, conv1_b, dense0_w, dense0_b, dense1_w, dense1_b, out_w, out_b):
    raise NotImplementedError("write your pallas kernel here")


# speedup vs baseline: 1.2414x; 1.2414x over previous
import functools

import jax
import jax.numpy as jnp
from jax.experimental import pallas as pl
from jax.experimental.pallas import tpu as pltpu


def _fused_kernel(x_ref, w0_ref, b0_ref, w1_ref, b1_ref, d0_ref, db0_ref,
                  d1_ref, db1_ref, ow_ref, ob_ref, o_ref, *, L, ck):
    tb = x_ref.shape[1]
    c_in = x_ref.shape[2]

    def mm(a, w):
        return jnp.dot(a, w, preferred_element_type=jnp.float32)

    h = x_ref[...].reshape(L * tb, c_in)
    zpad = jnp.zeros((tb, c_in), jnp.bfloat16)
    hp = jnp.concatenate([zpad, h, zpad], axis=0)
    a0 = jnp.concatenate([hp[0:L * tb],
                          hp[tb:(L + 1) * tb],
                          hp[2 * tb:(L + 2) * tb]], axis=1)
    y = jnp.maximum(mm(a0, w0_ref[...]) + b0_ref[...], 0.0)
    y = y.astype(jnp.bfloat16)

    c1 = y.shape[1]
    zpad1 = jnp.zeros((tb, c1), jnp.bfloat16)
    hp1 = jnp.concatenate([zpad1, y, zpad1], axis=0)
    a1 = jnp.concatenate([hp1[0:L * tb],
                          hp1[tb:(L + 1) * tb],
                          hp1[2 * tb:(L + 2) * tb]], axis=1)
    y2 = jnp.maximum(mm(a1, w1_ref[...]) + b1_ref[...], 0.0)
    y2 = y2.astype(jnp.bfloat16)

    acc = None
    for j in range(L // ck):
        seg = jnp.concatenate(
            [y2[(j * ck + t) * tb:(j * ck + t + 1) * tb] for t in range(ck)],
            axis=1)
        p = mm(seg, d0_ref[j])
        acc = p if acc is None else acc + p
    z = jnp.maximum(acc + db0_ref[...], 0.0).astype(jnp.bfloat16)

    z = jnp.maximum(mm(z, d1_ref[...]) + db1_ref[...], 0.0)
    z = z.astype(jnp.bfloat16)
    o_ref[...] = mm(z, ow_ref[...]) + ob_ref[...]


def kernel(x, conv0_w, conv0_b, conv1_w, conv1_b,
           dense0_w, dense0_b, dense1_w, dense1_b, out_w, out_b):
    B, c_in, L = x.shape
    c1 = conv1_w.shape[1]
    n_out = out_w.shape[1]

    tb = 256
    while B % tb:
        tb //= 2
    ck = 8
    while L % ck:
        ck //= 2

    xt = jnp.transpose(x, (2, 0, 1)).astype(jnp.bfloat16)
    w0 = conv0_w.astype(jnp.bfloat16)
    w1 = conv1_w.astype(jnp.bfloat16)
    d0 = dense0_w.reshape(L // ck, ck * c1, dense0_w.shape[1]).astype(jnp.bfloat16)
    d1 = dense1_w.astype(jnp.bfloat16)
    ow = out_w.astype(jnp.bfloat16)

    grid = (B // tb,)

    def bcast(arr):
        return pl.BlockSpec(arr.shape, lambda b: (0,) * arr.ndim)

    in_specs = [pl.BlockSpec((L, tb, c_in), lambda b: (0, b, 0)),
                bcast(w0), bcast(conv0_b), bcast(w1), bcast(conv1_b),
                bcast(d0), bcast(dense0_b), bcast(d1), bcast(dense1_b),
                bcast(ow), bcast(out_b)]
    out_specs = pl.BlockSpec((tb, n_out), lambda b: (b, 0))

    flops = 2 * B * L * (3 * c_in * conv0_w.shape[1] + 3 * c1 * c1) \
        + 2 * B * (L * c1 * dense0_w.shape[1]
                   + dense1_w.shape[0] * dense1_w.shape[1]
                   + out_w.shape[0] * n_out)
    weights = [w0, conv0_b, w1, conv1_b, d0, dense0_b, d1, dense1_b, ow, out_b]
    param_bytes = sum(int(a.size) * a.dtype.itemsize for a in weights)
    bytes_accessed = int(xt.size) * 2 + param_bytes + B * n_out * 4
    cost = pl.CostEstimate(flops=int(flops), transcendentals=0,
                           bytes_accessed=int(bytes_accessed))

    kern = functools.partial(_fused_kernel, L=L, ck=ck)
    return pl.pallas_call(
        kern,
        out_shape=jax.ShapeDtypeStruct((B, n_out), jnp.float32),
        grid=grid,
        in_specs=in_specs,
        out_specs=out_specs,
        compiler_params=pltpu.CompilerParams(
            dimension_semantics=("parallel",),
            vmem_limit_bytes=100 * 1024 * 1024),
        cost_estimate=cost,
    )(xt, w0, conv0_b, w1, conv1_b, d0, dense0_b, d1, dense1_b, ow, out_b)
